# Initial kernel scaffold; baseline (speedup 1.0000x reference)
import jax
import jax.numpy as jnp
from jax.experimental import pallas as pl


def kernel(hidden_states, W, scale, per_expert_scale):
    raise NotImplementedError("write your pallas kernel here")


# speedup vs baseline: 2.5956x; 2.5956x over previous
import jax
import jax.numpy as jnp
from jax.experimental import pallas as pl
from jax.experimental.pallas import tpu as pltpu

H = 2048
E = 64
EPS = 1e-06
TILE = 512


def _router_body(x_ref, w_ref, scale_ref, pes_ref, tw_ref, ti_ref):
    x = x_ref[...]
    h = x * jax.lax.rsqrt(jnp.mean(x * x, axis=1, keepdims=True) + EPS)
    h = h * (scale_ref[...] * (float(H) ** -0.5))
    logits = jax.lax.dot_general(
        h, w_ref[...], (((1,), (1,)), ((), ())),
        preferred_element_type=jnp.float32)

    idx = jax.lax.broadcasted_iota(jnp.int32, logits.shape, 1)
    m1 = jnp.max(logits, axis=1, keepdims=True)
    i1 = jnp.min(jnp.where(logits == m1, idx, E), axis=1, keepdims=True)
    masked = jnp.where(idx == i1, jnp.finfo(jnp.float32).min, logits)
    m2 = jnp.max(masked, axis=1, keepdims=True)
    i2 = jnp.min(jnp.where(masked == m2, idx, E), axis=1, keepdims=True)

    e = jnp.exp(m2 - m1)
    denom = 1.0 + e
    pes = pes_ref[...]
    s1 = jnp.sum(jnp.where(idx == i1, pes, 0.0), axis=1, keepdims=True)
    s2 = jnp.sum(jnp.where(idx == i2, pes, 0.0), axis=1, keepdims=True)
    tw_ref[:, 0:1] = s1 / denom
    tw_ref[:, 1:2] = (e / denom) * s2
    ti_ref[:, 0:1] = i1
    ti_ref[:, 1:2] = i2


def kernel(hidden_states, W, scale, per_expert_scale):
    T = hidden_states.shape[0]
    grid = (T // TILE,)
    scale2d = scale.reshape(1, H)
    pes2d = per_expert_scale.reshape(1, E)
    top_w, top_i = pl.pallas_call(
        _router_body,
        grid=grid,
        in_specs=[
            pl.BlockSpec((TILE, H), lambda i: (i, 0)),
            pl.BlockSpec((E, H), lambda i: (0, 0)),
            pl.BlockSpec((1, H), lambda i: (0, 0)),
            pl.BlockSpec((1, E), lambda i: (0, 0)),
        ],
        out_specs=[
            pl.BlockSpec((TILE, 2), lambda i: (i, 0)),
            pl.BlockSpec((TILE, 2), lambda i: (i, 0)),
        ],
        out_shape=[
            jax.ShapeDtypeStruct((T, 2), jnp.float32),
            jax.ShapeDtypeStruct((T, 2), jnp.int32),
        ],
        compiler_params=pltpu.CompilerParams(
            dimension_semantics=("arbitrary",),
        ),
    )(hidden_states, W, scale2d, pes2d)
    return (top_w, top_i)


# speedup vs baseline: 2.6163x; 1.0080x over previous
import jax
import jax.numpy as jnp
from jax.experimental import pallas as pl
from jax.experimental.pallas import tpu as pltpu

H = 2048
E = 64
EPS = 1e-06
TILE = 512


def _router_body(x_ref, w_ref, scale_ref, pes_ref, tw_ref, ti_ref, ws_ref):
    @pl.when(pl.program_id(0) == 0)
    def _():
        ws_ref[...] = w_ref[...] * (scale_ref[...] * (float(H) ** -0.5))

    x = x_ref[...]
    h = x * jax.lax.rsqrt(jnp.mean(x * x, axis=1, keepdims=True) + EPS)
    logits = jax.lax.dot_general(
        h, ws_ref[...], (((1,), (1,)), ((), ())),
        preferred_element_type=jnp.float32)

    idx = jax.lax.broadcasted_iota(jnp.int32, logits.shape, 1)
    m1 = jnp.max(logits, axis=1, keepdims=True)
    i1 = jnp.min(jnp.where(logits == m1, idx, E), axis=1, keepdims=True)
    masked = jnp.where(idx == i1, jnp.finfo(jnp.float32).min, logits)
    m2 = jnp.max(masked, axis=1, keepdims=True)
    i2 = jnp.min(jnp.where(masked == m2, idx, E), axis=1, keepdims=True)

    e = jnp.exp(m2 - m1)
    denom = 1.0 + e
    pes = pes_ref[...]
    s1 = jnp.sum(jnp.where(idx == i1, pes, 0.0), axis=1, keepdims=True)
    s2 = jnp.sum(jnp.where(idx == i2, pes, 0.0), axis=1, keepdims=True)
    tw_ref[:, 0:1] = s1 / denom
    tw_ref[:, 1:2] = (e / denom) * s2
    ti_ref[:, 0:1] = i1
    ti_ref[:, 1:2] = i2


def kernel(hidden_states, W, scale, per_expert_scale):
    T = hidden_states.shape[0]
    grid = (T // TILE,)
    scale2d = scale.reshape(1, H)
    pes2d = per_expert_scale.reshape(1, E)
    top_w, top_i = pl.pallas_call(
        _router_body,
        grid=grid,
        in_specs=[
            pl.BlockSpec((TILE, H), lambda i: (i, 0)),
            pl.BlockSpec((E, H), lambda i: (0, 0)),
            pl.BlockSpec((1, H), lambda i: (0, 0)),
            pl.BlockSpec((1, E), lambda i: (0, 0)),
        ],
        out_specs=[
            pl.BlockSpec((TILE, 2), lambda i: (i, 0)),
            pl.BlockSpec((TILE, 2), lambda i: (i, 0)),
        ],
        out_shape=[
            jax.ShapeDtypeStruct((T, 2), jnp.float32),
            jax.ShapeDtypeStruct((T, 2), jnp.int32),
        ],
        scratch_shapes=[pltpu.VMEM((E, H), jnp.float32)],
        compiler_params=pltpu.CompilerParams(
            dimension_semantics=("arbitrary",),
        ),
    )(hidden_states, W, scale2d, pes2d)
    return (top_w, top_i)


# speedup vs baseline: 3.0376x; 1.1610x over previous
import jax
import jax.numpy as jnp
from jax.experimental import pallas as pl
from jax.experimental.pallas import tpu as pltpu

H = 2048
E = 64
EPS = 1e-06
TILE = 1024


def _router_body(x_ref, w_ref, scale_ref, pes_ref, tw_ref, ti_ref):
    x = x_ref[...]
    h = x * jax.lax.rsqrt(jnp.mean(x * x, axis=1, keepdims=True) + EPS)
    h = h * (scale_ref[...] * (float(H) ** -0.5))
    logits = jax.lax.dot_general(
        h, w_ref[...], (((1,), (1,)), ((), ())),
        preferred_element_type=jnp.float32)

    idx = jax.lax.broadcasted_iota(jnp.int32, logits.shape, 1)
    m1 = jnp.max(logits, axis=1, keepdims=True)
    i1 = jnp.min(jnp.where(logits == m1, idx, E), axis=1, keepdims=True)
    masked = jnp.where(idx == i1, jnp.finfo(jnp.float32).min, logits)
    m2 = jnp.max(masked, axis=1, keepdims=True)
    i2 = jnp.min(jnp.where(masked == m2, idx, E), axis=1, keepdims=True)

    e = jnp.exp(m2 - m1)
    denom = 1.0 + e
    pes = pes_ref[...]
    s1 = jnp.sum(jnp.where(idx == i1, pes, 0.0), axis=1, keepdims=True)
    s2 = jnp.sum(jnp.where(idx == i2, pes, 0.0), axis=1, keepdims=True)
    tw_ref[:, 0:1] = s1 / denom
    tw_ref[:, 1:2] = (e / denom) * s2
    ti_ref[:, 0:1] = i1
    ti_ref[:, 1:2] = i2


def kernel(hidden_states, W, scale, per_expert_scale):
    T = hidden_states.shape[0]
    grid = (T // TILE,)
    scale2d = scale.reshape(1, H)
    pes2d = per_expert_scale.reshape(1, E)
    top_w, top_i = pl.pallas_call(
        _router_body,
        grid=grid,
        in_specs=[
            pl.BlockSpec((TILE, H), lambda i: (i, 0)),
            pl.BlockSpec((E, H), lambda i: (0, 0)),
            pl.BlockSpec((1, H), lambda i: (0, 0)),
            pl.BlockSpec((1, E), lambda i: (0, 0)),
        ],
        out_specs=[
            pl.BlockSpec((TILE, 2), lambda i: (i, 0)),
            pl.BlockSpec((TILE, 2), lambda i: (i, 0)),
        ],
        out_shape=[
            jax.ShapeDtypeStruct((T, 2), jnp.float32),
            jax.ShapeDtypeStruct((T, 2), jnp.int32),
        ],
        compiler_params=pltpu.CompilerParams(
            dimension_semantics=("arbitrary",),
        ),
    )(hidden_states, W, scale2d, pes2d)
    return (top_w, top_i)


# speedup vs baseline: 3.1392x; 1.0334x over previous
import jax
import jax.numpy as jnp
from jax.experimental import pallas as pl
from jax.experimental.pallas import tpu as pltpu

H = 2048
E = 64
EPS = 1e-06
TILE = 2048


def _router_body(x_ref, w_ref, scale_ref, pes_ref, tw_ref, ti_ref):
    x = x_ref[...]
    h = x * jax.lax.rsqrt(jnp.mean(x * x, axis=1, keepdims=True) + EPS)
    h = h * (scale_ref[...] * (float(H) ** -0.5))
    logits = jax.lax.dot_general(
        h, w_ref[...], (((1,), (1,)), ((), ())),
        preferred_element_type=jnp.float32)

    idx = jax.lax.broadcasted_iota(jnp.int32, logits.shape, 1)
    m1 = jnp.max(logits, axis=1, keepdims=True)
    i1 = jnp.min(jnp.where(logits == m1, idx, E), axis=1, keepdims=True)
    masked = jnp.where(idx == i1, jnp.finfo(jnp.float32).min, logits)
    m2 = jnp.max(masked, axis=1, keepdims=True)
    i2 = jnp.min(jnp.where(masked == m2, idx, E), axis=1, keepdims=True)

    e = jnp.exp(m2 - m1)
    denom = 1.0 + e
    pes = pes_ref[...]
    s1 = jnp.sum(jnp.where(idx == i1, pes, 0.0), axis=1, keepdims=True)
    s2 = jnp.sum(jnp.where(idx == i2, pes, 0.0), axis=1, keepdims=True)
    tw_ref[:, 0:1] = s1 / denom
    tw_ref[:, 1:2] = (e / denom) * s2
    ti_ref[:, 0:1] = i1
    ti_ref[:, 1:2] = i2


def kernel(hidden_states, W, scale, per_expert_scale):
    T = hidden_states.shape[0]
    grid = (T // TILE,)
    scale2d = scale.reshape(1, H)
    pes2d = per_expert_scale.reshape(1, E)
    top_w, top_i = pl.pallas_call(
        _router_body,
        grid=grid,
        in_specs=[
            pl.BlockSpec((TILE, H), lambda i: (i, 0)),
            pl.BlockSpec((E, H), lambda i: (0, 0)),
            pl.BlockSpec((1, H), lambda i: (0, 0)),
            pl.BlockSpec((1, E), lambda i: (0, 0)),
        ],
        out_specs=[
            pl.BlockSpec((TILE, 2), lambda i: (i, 0)),
            pl.BlockSpec((TILE, 2), lambda i: (i, 0)),
        ],
        out_shape=[
            jax.ShapeDtypeStruct((T, 2), jnp.float32),
            jax.ShapeDtypeStruct((T, 2), jnp.int32),
        ],
        compiler_params=pltpu.CompilerParams(
            dimension_semantics=("arbitrary",),
        ),
    )(hidden_states, W, scale2d, pes2d)
    return (top_w, top_i)
